# baseline (speedup 1.0000x reference)
import functools

import jax
import jax.numpy as jnp
from jax import lax
from jax.experimental import pallas as pl
from jax.experimental.pallas import tpu as pltpu
from jax.experimental.pallas import tpu_sc as plsc

NC = 2
NS = 16
NW = NC * NS
LANES = 16
CHUNK = 128


@functools.lru_cache(maxsize=None)
def _make_sc_kernel(B, D, b_per_w):
    assert D == 2 * LANES
    n_chunks = b_per_w // CHUNK
    mesh = plsc.VectorSubcoreMesh(core_axis_name="c", subcore_axis_name="s")

    @functools.partial(
        pl.kernel,
        out_type=jax.ShapeDtypeStruct((B,), jnp.float32),
        mesh=mesh,
        scratch_types=[
            pltpu.VMEM((n_chunks, CHUNK), jnp.int32),
            pltpu.VMEM((n_chunks, CHUNK), jnp.int32),
            pltpu.VMEM((b_per_w, D), jnp.float32),
            pltpu.VMEM((b_per_w, D), jnp.float32),
            pltpu.VMEM((b_per_w, LANES), jnp.float32),
            pltpu.VMEM((b_per_w,), jnp.float32),
            pltpu.SemaphoreType.DMA,
            pltpu.SemaphoreType.DMA,
        ],
        compiler_params=pltpu.CompilerParams(
            needs_layout_passes=False, use_tc_tiling_on_sc=False),
    )
    def k(uids_hbm, iids_hbm, ut_hbm, it_hbm, out_hbm,
          uidx_v, iidx_v, urows_v, irows_v, part_v, out_v, sem_u, sem_i):
        wid = lax.axis_index("s") * NC + lax.axis_index("c")
        base = wid * b_per_w

        pltpu.sync_copy(uids_hbm.at[pl.ds(wid * n_chunks, n_chunks)], uidx_v)
        pltpu.sync_copy(iids_hbm.at[pl.ds(wid * n_chunks, n_chunks)], iidx_v)

        copies = []
        for j in range(n_chunks):
            copies.append(pltpu.async_copy(
                ut_hbm.at[uidx_v.at[j]],
                urows_v.at[pl.ds(j * CHUNK, CHUNK)], sem_u))
            copies.append(pltpu.async_copy(
                it_hbm.at[iidx_v.at[j]],
                irows_v.at[pl.ds(j * CHUNK, CHUNK)], sem_i))
        for c in copies:
            c.wait()

        lane = lax.iota(jnp.int32, LANES)

        def group(g, carry):
            def row(i, acc):
                j = g * LANES + i
                u0 = urows_v[j, pl.ds(0, LANES)]
                u1 = urows_v[j, pl.ds(LANES, LANES)]
                i0 = irows_v[j, pl.ds(0, LANES)]
                i1 = irows_v[j, pl.ds(LANES, LANES)]
                s = jnp.sum(u0 * i0 + u1 * i1)
                return jnp.where(lane == i, s, acc)
            acc = lax.fori_loop(0, LANES, row, jnp.zeros((LANES,), jnp.float32))
            out_v[pl.ds(g * LANES, LANES)] = acc
            return carry
        lax.fori_loop(0, b_per_w // LANES, group, 0)

        pltpu.sync_copy(out_v, out_hbm.at[pl.ds(base, b_per_w)])

    return k


def kernel(user_ids, item_ids, user_table, item_table):
    B = user_ids.shape[0]
    D = user_table.shape[1]
    b_per_w = B // NW
    k = _make_sc_kernel(B, D, b_per_w)
    uids2 = user_ids.reshape(B // CHUNK, CHUNK)
    iids2 = item_ids.reshape(B // CHUNK, CHUNK)
    return k(uids2, iids2, user_table, item_table)
